# baseline (speedup 1.0000x reference)
import functools
import math

import jax
import jax.numpy as jnp
from jax import lax
from jax.experimental import pallas as pl
from jax.experimental.pallas import tpu as pltpu
from jax.experimental.pallas import tpu_sc as plsc

_SMOOTHING = 0.1
_CONFIDENCE = 1.0 - _SMOOTHING
_PAD = 0
_BLOCK_W = 4096

_NC, _NS, _L = 2, 16, 16
_NW = _NC * _NS


def _sc_gather(x_flat, target, v):
    b = target.shape[0]
    bpw = b // _NW
    mesh = plsc.VectorSubcoreMesh(
        core_axis_name="c", subcore_axis_name="s",
        num_cores=_NC, num_subcores=_NS)

    @functools.partial(
        pl.kernel,
        out_type=jax.ShapeDtypeStruct((b,), jnp.float32),
        mesh=mesh,
        scratch_types=[
            pltpu.VMEM((bpw,), jnp.int32),
            pltpu.VMEM((bpw,), jnp.int32),
            pltpu.VMEM((bpw,), jnp.float32),
            pltpu.SemaphoreType.DMA,
        ],
    )
    def k(x_hbm, tgt_hbm, out_hbm, tgt_v, idx_v, g_v, sem):
        wid = lax.axis_index("s") * _NC + lax.axis_index("c")
        base = wid * bpw
        pltpu.sync_copy(tgt_hbm.at[pl.ds(base, bpw)], tgt_v)
        for c in range(bpw // _L):
            t = tgt_v[pl.ds(c * _L, _L)]
            rows = lax.broadcasted_iota(jnp.int32, (_L,), 0) + (base + c * _L)
            idx_v[pl.ds(c * _L, _L)] = rows * v + t
        pltpu.async_copy(x_hbm.at[idx_v], g_v, sem).wait()
        pltpu.sync_copy(g_v, out_hbm.at[pl.ds(base, bpw)])

    return k(x_flat, target)


def _tc_body(batch, v, block_w, nblocks, eps, c1):
    def body(target_ref, g_ref, x_ref, out_ref, s_acc, x0_acc):
        j = pl.program_id(0)

        @pl.when(j == 0)
        def _init():
            s_acc[:, :] = jnp.zeros_like(s_acc)
            x0_acc[:, :] = x_ref[:, 0:1]

        @pl.when(j < nblocks - 1)
        def _interior():
            s_acc[:, :] += jnp.sum(x_ref[:, :], axis=1, keepdims=True)

        @pl.when(j == nblocks - 1)
        def _final():
            cols = lax.broadcasted_iota(jnp.int32, (batch, block_w), 1) + j * block_w
            xm = jnp.where(cols < v, x_ref[:, :], 0.0)
            s = s_acc[:, :] + jnp.sum(xm, axis=1, keepdims=True)
            valid = (target_ref[:, :] != _PAD).astype(jnp.float32)
            per_row = valid * (c1 - eps * s + eps * x0_acc[:, :]
                               + (eps - _CONFIDENCE) * g_ref[:, :])
            out_ref[:, :] = jnp.sum(per_row, keepdims=True)

    return body


def kernel(x, target):
    batch, v = x.shape
    eps = _SMOOTHING / (v - 2)
    c1 = eps * math.log(eps) * (v - 2) + _CONFIDENCE * math.log(_CONFIDENCE)
    nblocks = (v + _BLOCK_W - 1) // _BLOCK_W

    t32 = target.astype(jnp.int32)
    g = _sc_gather(x.reshape(-1), t32, v)

    out = pl.pallas_call(
        _tc_body(batch, v, _BLOCK_W, nblocks, eps, c1),
        grid=(nblocks,),
        in_specs=[
            pl.BlockSpec((batch, 1), lambda j: (0, 0)),
            pl.BlockSpec((batch, 1), lambda j: (0, 0)),
            pl.BlockSpec((batch, _BLOCK_W), lambda j: (0, j)),
        ],
        out_specs=pl.BlockSpec((1, 1), lambda j: (0, 0)),
        out_shape=jax.ShapeDtypeStruct((1, 1), jnp.float32),
        scratch_shapes=[
            pltpu.VMEM((batch, 1), jnp.float32),
            pltpu.VMEM((batch, 1), jnp.float32),
        ],
    )(t32.reshape(batch, 1), g.reshape(batch, 1), x)
    return out[0, 0]


# speedup vs baseline: 1.0118x; 1.0118x over previous
import functools
import math

import jax
import jax.numpy as jnp
from jax import lax
from jax.experimental import pallas as pl
from jax.experimental.pallas import tpu as pltpu
from jax.experimental.pallas import tpu_sc as plsc

_SMOOTHING = 0.1
_CONFIDENCE = 1.0 - _SMOOTHING
_PAD = 0
_BLOCK_W = 4096

_NC, _NS, _L = 2, 16, 16
_NW = _NC * _NS


def _sc_gather(x_flat, target, v):
    b = target.shape[0]
    bpw = b // _NW
    mesh = plsc.VectorSubcoreMesh(
        core_axis_name="c", subcore_axis_name="s",
        num_cores=_NC, num_subcores=_NS)

    @functools.partial(
        pl.kernel,
        out_type=jax.ShapeDtypeStruct((b,), jnp.float32),
        mesh=mesh,
        scratch_types=[
            pltpu.VMEM((bpw,), jnp.int32),
            pltpu.VMEM((bpw,), jnp.int32),
            pltpu.VMEM((bpw,), jnp.float32),
            pltpu.SemaphoreType.DMA,
        ],
    )
    def k(x_hbm, tgt_hbm, out_hbm, tgt_v, idx_v, g_v, sem):
        wid = lax.axis_index("s") * _NC + lax.axis_index("c")
        base = wid * bpw
        pltpu.sync_copy(tgt_hbm.at[pl.ds(base, bpw)], tgt_v)
        for c in range(bpw // _L):
            t = tgt_v[pl.ds(c * _L, _L)]
            rows = lax.broadcasted_iota(jnp.int32, (_L,), 0) + (base + c * _L)
            idx_v[pl.ds(c * _L, _L)] = rows * v + t
        pltpu.async_copy(x_hbm.at[idx_v], g_v, sem).wait()
        pltpu.sync_copy(g_v, out_hbm.at[pl.ds(base, bpw)])

    return k(x_flat, target)


def _tc_body(block_r, v, eps, c1):
    def body(target_ref, g_ref, x_ref, out_ref):
        i = pl.program_id(0)
        s = jnp.sum(x_ref[:, :], axis=1, keepdims=True)
        x0 = x_ref[:, 0:1]
        valid = (target_ref[:, :] != _PAD).astype(jnp.float32)
        per_row = valid * (c1 - eps * s + eps * x0
                           + (eps - _CONFIDENCE) * g_ref[:, :])
        partial = jnp.sum(per_row, keepdims=True)

        @pl.when(i == 0)
        def _init():
            out_ref[:, :] = jnp.zeros_like(out_ref)

        out_ref[:, :] += partial

    return body


_BLOCK_R = 32


def kernel(x, target):
    batch, v = x.shape
    eps = _SMOOTHING / (v - 2)
    c1 = eps * math.log(eps) * (v - 2) + _CONFIDENCE * math.log(_CONFIDENCE)
    nblocks = batch // _BLOCK_R

    t32 = target.astype(jnp.int32)
    g = _sc_gather(x.reshape(-1), t32, v)

    out = pl.pallas_call(
        _tc_body(_BLOCK_R, v, eps, c1),
        grid=(nblocks,),
        in_specs=[
            pl.BlockSpec((_BLOCK_R, 1), lambda i: (i, 0)),
            pl.BlockSpec((_BLOCK_R, 1), lambda i: (i, 0)),
            pl.BlockSpec((_BLOCK_R, v), lambda i: (i, 0)),
        ],
        out_specs=pl.BlockSpec((1, 1), lambda i: (0, 0)),
        out_shape=jax.ShapeDtypeStruct((1, 1), jnp.float32),
    )(t32.reshape(batch, 1), g.reshape(batch, 1), x)
    return out[0, 0]


# speedup vs baseline: 2.2251x; 2.1992x over previous
import functools
import math

import jax
import jax.numpy as jnp
from jax import lax
from jax.experimental import pallas as pl
from jax.experimental.pallas import tpu as pltpu
from jax.experimental.pallas import tpu_sc as plsc

_SMOOTHING = 0.1
_CONFIDENCE = 1.0 - _SMOOTHING
_PAD = 0
_BLOCK_W = 4096

_NC, _NS, _L = 2, 16, 16
_NW = _NC * _NS


def _sc_gather(x_flat, target, v):
    b = target.shape[0]
    bpw = b // _NW
    mesh = plsc.VectorSubcoreMesh(
        core_axis_name="c", subcore_axis_name="s",
        num_cores=_NC, num_subcores=_NS)

    @functools.partial(
        pl.kernel,
        out_type=jax.ShapeDtypeStruct((b,), jnp.float32),
        mesh=mesh,
        scratch_types=[
            pltpu.VMEM((bpw,), jnp.int32),
            pltpu.VMEM((bpw,), jnp.int32),
            pltpu.VMEM((bpw,), jnp.float32),
            pltpu.SemaphoreType.DMA,
        ],
    )
    def k(x_hbm, tgt_hbm, out_hbm, tgt_v, idx_v, g_v, sem):
        wid = lax.axis_index("s") * _NC + lax.axis_index("c")
        base = wid * bpw
        pltpu.sync_copy(tgt_hbm.at[pl.ds(base, bpw)], tgt_v)
        for c in range(bpw // _L):
            t = tgt_v[pl.ds(c * _L, _L)]
            rows = lax.broadcasted_iota(jnp.int32, (_L,), 0) + (base + c * _L)
            idx_v[pl.ds(c * _L, _L)] = rows * v + t
        pltpu.async_copy(x_hbm.at[idx_v], g_v, sem).wait()
        pltpu.sync_copy(g_v, out_hbm.at[pl.ds(base, bpw)])

    return k(x_flat, target)


def _tc_body(block_r, v, eps, c1):
    def body(target_ref, x_ref, out_ref):
        i = pl.program_id(0)
        xv = x_ref[:, :]
        t = target_ref[:, :]
        cols = lax.broadcasted_iota(jnp.int32, (block_r, v), 1)
        s = jnp.sum(xv, axis=1, keepdims=True)
        g = jnp.sum(jnp.where(cols == t, xv, 0.0), axis=1, keepdims=True)
        x0 = xv[:, 0:1]
        valid = (t != _PAD).astype(jnp.float32)
        per_row = valid * (c1 - eps * s + eps * x0
                           + (eps - _CONFIDENCE) * g)
        partial = jnp.sum(per_row, keepdims=True)

        @pl.when(i == 0)
        def _init():
            out_ref[:, :] = jnp.zeros_like(out_ref)

        out_ref[:, :] += partial

    return body


_BLOCK_R = 32


def kernel(x, target):
    batch, v = x.shape
    eps = _SMOOTHING / (v - 2)
    c1 = eps * math.log(eps) * (v - 2) + _CONFIDENCE * math.log(_CONFIDENCE)
    nblocks = batch // _BLOCK_R

    t32 = target.astype(jnp.int32)

    out = pl.pallas_call(
        _tc_body(_BLOCK_R, v, eps, c1),
        grid=(nblocks,),
        in_specs=[
            pl.BlockSpec((_BLOCK_R, 1), lambda i: (i, 0)),
            pl.BlockSpec((_BLOCK_R, v), lambda i: (i, 0)),
        ],
        out_specs=pl.BlockSpec((1, 1), lambda i: (0, 0)),
        out_shape=jax.ShapeDtypeStruct((1, 1), jnp.float32),
    )(t32.reshape(batch, 1), x)
    return out[0, 0]
